# baseline (speedup 1.0000x reference)
import functools

import jax
import jax.numpy as jnp
from jax import lax
from jax.experimental import pallas as pl
from jax.experimental.pallas import tpu as pltpu, tpu_sc as plsc

N = 10000
P = 5000
E = 320000
D = 128

NC, NS, L = 2, 16, 16
NW = NC * NS
TRI_W = 384
N_PAD = NW * TRI_W
NV = 10240
EPW = E // NW
B = 40
NCHUNK = EPW // B
BATCH = 10
NBATCH = NCHUNK // BATCH
FW = BATCH * 2 * B
DC = D + 16
DCP = 256
RPT = N // NS
ZR = 25

_mesh = plsc.VectorSubcoreMesh(
    core_axis_name="c", subcore_axis_name="s", num_cores=NC, num_subcores=NS)


def _geom_body(px_hbm, py_hbm, pz_hbm, t0_hbm, t1_hbm, t2_hbm, g_hbm,
               px_v, py_v, pz_v, t0_v, t1_v, t2_v, gt_v):
    c = lax.axis_index("c")
    s = lax.axis_index("s")
    wid = s * NC + c
    base = pl.multiple_of(wid * TRI_W, 128)
    pltpu.sync_copy(px_hbm, px_v)
    pltpu.sync_copy(py_hbm, py_v)
    pltpu.sync_copy(pz_hbm, pz_v)
    pltpu.sync_copy(t0_hbm.at[pl.ds(base, TRI_W)], t0_v)
    pltpu.sync_copy(t1_hbm.at[pl.ds(base, TRI_W)], t1_v)
    pltpu.sync_copy(t2_hbm.at[pl.ds(base, TRI_W)], t2_v)

    zero = jnp.zeros((L,), jnp.float32)
    third = jnp.full((L,), 1.0 / 3.0, jnp.float32)
    coords = (px_v, py_v, pz_v)
    tcols = (t0_v, t1_v, t2_v)

    def group(i, _):
        sl = pl.ds(i * L, L)
        idx = [tcols[j][sl] for j in range(3)]
        for cc in range(3):
            p0 = plsc.load_gather(coords[cc], [idx[0]])
            p1 = plsc.load_gather(coords[cc], [idx[1]])
            p2 = plsc.load_gather(coords[cc], [idx[2]])
            e0 = p0 - p1
            e1 = p0 - p2
            e2 = p1 - p2
            gt_v[0 + cc, sl] = jnp.minimum(jnp.minimum(e0, e1), e2)
            gt_v[3 + cc, sl] = jnp.maximum(jnp.maximum(e0, e1), e2)
            gt_v[6 + cc, sl] = (p0 + p1 + p2) * third
        for r in range(9, 16):
            gt_v[r, sl] = zero
        return 0

    lax.fori_loop(0, TRI_W // L, group, 0)
    pltpu.sync_copy(gt_v, g_hbm.at[:, pl.ds(base, TRI_W)])


_geom = functools.partial(
    pl.kernel,
    out_type=jax.ShapeDtypeStruct((16, N_PAD), jnp.float32),
    mesh=_mesh,
    scratch_types=[
        pltpu.VMEM((P,), jnp.float32),
        pltpu.VMEM((P,), jnp.float32),
        pltpu.VMEM((P,), jnp.float32),
        pltpu.VMEM((TRI_W,), jnp.int32),
        pltpu.VMEM((TRI_W,), jnp.int32),
        pltpu.VMEM((TRI_W,), jnp.int32),
        pltpu.VMEM((16, TRI_W), jnp.float32),
    ],
    compiler_params=pltpu.CompilerParams(needs_layout_passes=False),
)(_geom_body)


def _edge_body(v_hbm, edges_hbm, b1_hbm, hc_hbm,
               h_sp,
               ib0, ib1, ic0, ic1, rows0, rows1, hr0, hr1, b1_v, zbuf,
               sem_f0, sem_f1, sem_g0, sem_g1, sem_c0, sem_c1):
    c = lax.axis_index("c")
    s = lax.axis_index("s")
    wid = s * NC + c
    cbase = wid * NCHUNK

    ib = (ib0, ib1)
    ic = (ic0, ic1)
    rows = (rows0, rows1)
    hr = (hr0, hr1)
    sem_f = (sem_f0, sem_f1)
    sem_g = (sem_g0, sem_g1)
    sem_c = (sem_c0, sem_c1)

    zero = jnp.zeros((L,), jnp.float32)

    def zrow(i, _):
        for r in range(DC // L):
            zbuf[i, pl.ds(r * L, L)] = zero
        return 0

    lax.fori_loop(0, ZR, zrow, 0)
    for m in range(RPT // ZR):
        pltpu.sync_copy(zbuf, h_sp.at[pl.ds(s * RPT + m * ZR, ZR)])
    pltpu.sync_copy(b1_hbm, b1_v)

    b1v = [b1_v[pl.ds(r * L, L)] for r in range(D // L)]
    lane = lax.iota(jnp.int32, L)
    cnt_one = jnp.where(lane == 0, 1.0, 0.0).astype(jnp.float32)

    def cinit(i, _):
        hr0[i, pl.ds(D, L)] = cnt_one
        hr1[i, pl.ds(D, L)] = cnt_one
        return 0

    lax.fori_loop(0, B, cinit, 0)

    def issue_fetch(m, fb):
        base = pl.multiple_of((cbase + m * BATCH) * 2 * B, 8)
        pltpu.async_copy(edges_hbm.at[pl.ds(base, FW)], ib[fb], sem_f[fb])

    def wait_fetch(fb):
        pltpu.make_async_copy(edges_hbm.at[pl.ds(0, FW)], ib[fb],
                              sem_f[fb]).wait()

    def issue_gather(fb, j, b):
        pltpu.async_copy(v_hbm.at[ib[fb].at[pl.ds(j * 2 * B, 2 * B)]],
                         rows[b], sem_g[b])

    def wait_gather(b):
        pltpu.make_async_copy(v_hbm.at[ib[0].at[pl.ds(0, 2 * B)]],
                              rows[b], sem_g[b]).wait()

    def copy_idx(fb, j, b):
        for o in (0, 16, B - 16):
            ic[b][pl.ds(o, L)] = ib[fb][pl.ds(j * 2 * B + o, L)]

    def issue_scatter(b):
        pltpu.async_copy(hr[b], h_sp.at[ic[b]], sem_c[b], add=True)

    def wait_scatter(b):
        pltpu.make_async_copy(hr[b], h_sp.at[ic[b]], sem_c[b]).wait()

    def compute(b):
        @plsc.parallel_loop(0, B, unroll=4)
        def edge(i):
            for r in range(D // L):
                sl = pl.ds(r * L, L)
                hr[b][i, sl] = jnp.maximum(
                    rows[b][i, sl] - rows[b][B + i, sl] + b1v[r], 0.0)

    issue_fetch(jnp.int32(0), 0)
    wait_fetch(0)
    issue_gather(0, 0, 0)
    plsc.subcore_barrier()

    def batch_body(m, fb):
        @pl.when(m < NBATCH - 1)
        def _():
            issue_fetch(m + 1, 1 - fb)

        def chunk(jj, j, jp):
            @pl.when((m > 0) | (jj > 0))
            def _():
                wait_scatter(jp)

            copy_idx(fb, j, jp)
            wait_gather(jp)

            @pl.when(j < BATCH - 1)
            def _():
                issue_gather(fb, j + 1, 1 - jp)

            @pl.when((j == BATCH - 1) & (m < NBATCH - 1))
            def _():
                wait_fetch(1 - fb)
                issue_gather(1 - fb, 0, 1 - jp)

            compute(jp)
            issue_scatter(jp)

        def pair(jj, _):
            chunk(jj, jj * 2, 0)
            chunk(jj, jj * 2 + 1, 1)
            return 0

        lax.fori_loop(0, BATCH // 2, pair, 0)

    def outer(bp, _):
        batch_body(bp * 2, 0)
        batch_body(bp * 2 + 1, 1)
        return 0

    lax.fori_loop(0, NBATCH // 2, outer, 0)
    batch_body(jnp.int32(NBATCH - 1), 0)

    wait_scatter(0)
    wait_scatter(1)
    plsc.subcore_barrier()

    @pl.when(s == 0)
    def _():
        pltpu.sync_copy(h_sp,
                        hc_hbm.at[pl.ds(c * N, N), pl.ds(0, DC)])


_edge = functools.partial(
    pl.kernel,
    out_type=jax.ShapeDtypeStruct((NC * N, DCP), jnp.float32),
    mesh=_mesh,
    scratch_types=[
        pltpu.VMEM_SHARED((N, DC), jnp.float32),
        pltpu.VMEM((FW,), jnp.int32),
        pltpu.VMEM((FW,), jnp.int32),
        pltpu.VMEM((B,), jnp.int32),
        pltpu.VMEM((B,), jnp.int32),
        pltpu.VMEM((2 * B, D), jnp.float32),
        pltpu.VMEM((2 * B, D), jnp.float32),
        pltpu.VMEM((B, DC), jnp.float32),
        pltpu.VMEM((B, DC), jnp.float32),
        pltpu.VMEM((D,), jnp.float32),
        pltpu.VMEM((ZR, DC), jnp.float32),
        pltpu.SemaphoreType.DMA,
        pltpu.SemaphoreType.DMA,
        pltpu.SemaphoreType.DMA,
        pltpu.SemaphoreType.DMA,
        pltpu.SemaphoreType.DMA,
        pltpu.SemaphoreType.DMA,
    ],
    compiler_params=pltpu.CompilerParams(
        needs_layout_passes=False, use_tc_tiling_on_sc=False),
)(_edge_body)


BN1 = 1024
BN2 = 1000


def _mm1_body(x_ref, g_ref, w1x_ref, w1g_ref, v_ref):
    v_ref[...] = (
        jnp.dot(x_ref[...], w1x_ref[...], preferred_element_type=jnp.float32)
        + lax.dot_general(
            g_ref[0], w1g_ref[...],
            dimension_numbers=(((0,), (0,)), ((), ())),
            preferred_element_type=jnp.float32))


def _mm1(x_pad, g3, w1x, w1g_p):
    return pl.pallas_call(
        _mm1_body,
        grid=(NV // BN1,),
        in_specs=[
            pl.BlockSpec((BN1, D), lambda j: (j, 0)),
            pl.BlockSpec((1, 16, BN1), lambda j: (j, 0, 0)),
            pl.BlockSpec((D, D), lambda j: (0, 0)),
            pl.BlockSpec((16, D), lambda j: (0, 0)),
        ],
        out_specs=pl.BlockSpec((BN1, D), lambda j: (j, 0)),
        out_shape=jax.ShapeDtypeStruct((NV, D), jnp.float32),
    )(x_pad, g3, w1x, w1g_p)


def _mm2_body(h0_ref, h1_ref, w2_ref, b2_ref, o_ref):
    a = h0_ref[:, :D] + h1_ref[:, :D]
    cnt = h0_ref[:, D:D + 1] + h1_ref[:, D:D + 1]
    o_ref[...] = (
        jnp.dot(a, w2_ref[...], preferred_element_type=jnp.float32)
        + cnt * b2_ref[...])


def _mm2(hc, w2, b2_row):
    return pl.pallas_call(
        _mm2_body,
        grid=(N // BN2,),
        in_specs=[
            pl.BlockSpec((BN2, DCP), lambda j: (j, 0)),
            pl.BlockSpec((BN2, DCP), lambda j: (j + N // BN2, 0)),
            pl.BlockSpec((D, D), lambda j: (0, 0)),
            pl.BlockSpec((1, D), lambda j: (0, 0)),
        ],
        out_specs=pl.BlockSpec((BN2, D), lambda j: (j, 0)),
        out_shape=jax.ShapeDtypeStruct((N, D), jnp.float32),
    )(hc, hc, w2, b2_row)


def kernel(x, points, triangles, triangle_edges, W1, b1, W2, b2):
    tri = triangles.astype(jnp.int32)
    tpad = jnp.zeros((N_PAD - N,), jnp.int32)
    t0 = jnp.concatenate([tri[:, 0], tpad])
    t1 = jnp.concatenate([tri[:, 1], tpad])
    t2 = jnp.concatenate([tri[:, 2], tpad])
    px, py, pz = points[:, 0], points[:, 1], points[:, 2]
    src = triangle_edges[0].astype(jnp.int32)
    tgt = triangle_edges[1].astype(jnp.int32)
    edges2 = jnp.concatenate(
        [src.reshape(E // B, B), tgt.reshape(E // B, B)], axis=1).reshape(-1)
    w1g_p = jnp.concatenate(
        [W1[:9], jnp.zeros((16 - 9, D), W1.dtype)], axis=0)
    w1x = W1[9:]
    x_pad = jnp.concatenate([x, jnp.zeros((NV - N, D), x.dtype)], axis=0)

    g_t = _geom(px, py, pz, t0, t1, t2)
    g3 = jnp.moveaxis(g_t.reshape(16, N_PAD // BN1, BN1), 1, 0)
    v = _mm1(x_pad, g3, w1x, w1g_p)
    hc = _edge(v, edges2, b1)
    return _mm2(hc, W2, b2.reshape(1, D))


# speedup vs baseline: 1.0472x; 1.0472x over previous
import functools

import jax
import jax.numpy as jnp
from jax import lax
from jax.experimental import pallas as pl
from jax.experimental.pallas import tpu as pltpu, tpu_sc as plsc

N = 10000
P = 5000
E = 320000
D = 128

NC, NS, L = 2, 16, 16
NW = NC * NS
TRI_W = 384
N_PAD = NW * TRI_W
NV = 10240
EPW = E // NW
B = 40
NCHUNK = EPW // B
DC = D + 16
DCP = 256
RPT = N // NS
ZR = 25

_mesh = plsc.VectorSubcoreMesh(
    core_axis_name="c", subcore_axis_name="s", num_cores=NC, num_subcores=NS)


def _geom_body(px_hbm, py_hbm, pz_hbm, t0_hbm, t1_hbm, t2_hbm, g_hbm,
               px_v, py_v, pz_v, t0_v, t1_v, t2_v, gt_v):
    c = lax.axis_index("c")
    s = lax.axis_index("s")
    wid = s * NC + c
    base = pl.multiple_of(wid * TRI_W, 128)
    pltpu.sync_copy(px_hbm, px_v)
    pltpu.sync_copy(py_hbm, py_v)
    pltpu.sync_copy(pz_hbm, pz_v)
    pltpu.sync_copy(t0_hbm.at[pl.ds(base, TRI_W)], t0_v)
    pltpu.sync_copy(t1_hbm.at[pl.ds(base, TRI_W)], t1_v)
    pltpu.sync_copy(t2_hbm.at[pl.ds(base, TRI_W)], t2_v)

    zero = jnp.zeros((L,), jnp.float32)
    third = jnp.full((L,), 1.0 / 3.0, jnp.float32)
    coords = (px_v, py_v, pz_v)
    tcols = (t0_v, t1_v, t2_v)

    def group(i, _):
        sl = pl.ds(i * L, L)
        idx = [tcols[j][sl] for j in range(3)]
        for cc in range(3):
            p0 = plsc.load_gather(coords[cc], [idx[0]])
            p1 = plsc.load_gather(coords[cc], [idx[1]])
            p2 = plsc.load_gather(coords[cc], [idx[2]])
            e0 = p0 - p1
            e1 = p0 - p2
            e2 = p1 - p2
            gt_v[0 + cc, sl] = jnp.minimum(jnp.minimum(e0, e1), e2)
            gt_v[3 + cc, sl] = jnp.maximum(jnp.maximum(e0, e1), e2)
            gt_v[6 + cc, sl] = (p0 + p1 + p2) * third
        for r in range(9, 16):
            gt_v[r, sl] = zero
        return 0

    lax.fori_loop(0, TRI_W // L, group, 0)
    pltpu.sync_copy(gt_v, g_hbm.at[:, pl.ds(base, TRI_W)])


_geom = functools.partial(
    pl.kernel,
    out_type=jax.ShapeDtypeStruct((16, N_PAD), jnp.float32),
    mesh=_mesh,
    scratch_types=[
        pltpu.VMEM((P,), jnp.float32),
        pltpu.VMEM((P,), jnp.float32),
        pltpu.VMEM((P,), jnp.float32),
        pltpu.VMEM((TRI_W,), jnp.int32),
        pltpu.VMEM((TRI_W,), jnp.int32),
        pltpu.VMEM((TRI_W,), jnp.int32),
        pltpu.VMEM((16, TRI_W), jnp.float32),
    ],
    compiler_params=pltpu.CompilerParams(needs_layout_passes=False),
)(_geom_body)


def _edge_body(vp_hbm, v_hbm, src_hbm, tgt_hbm, hc_hbm,
               h_sp,
               idx_s0, idx_s1, idx_t0, idx_t1, idx_c0, idx_c1,
               rows_s0, rows_s1, rows_t0, rows_t1, hrows0, hrows1,
               zbuf,
               sem_i0, sem_i1, sem_g0, sem_g1, sem_c0, sem_c1):
    c = lax.axis_index("c")
    s = lax.axis_index("s")
    wid = s * NC + c
    ebase = wid * EPW

    idx_s = (idx_s0, idx_s1)
    idx_t = (idx_t0, idx_t1)
    idx_c = (idx_c0, idx_c1)
    rows_s = (rows_s0, rows_s1)
    rows_t = (rows_t0, rows_t1)
    hrows = (hrows0, hrows1)
    sem_i = (sem_i0, sem_i1)
    sem_g = (sem_g0, sem_g1)
    sem_c = (sem_c0, sem_c1)

    zero = jnp.zeros((L,), jnp.float32)

    def zrow(i, _):
        for r in range(DC // L):
            zbuf[i, pl.ds(r * L, L)] = zero
        return 0

    lax.fori_loop(0, ZR, zrow, 0)
    for m in range(RPT // ZR):
        pltpu.sync_copy(zbuf, h_sp.at[pl.ds(s * RPT + m * ZR, ZR)])

    lane = lax.iota(jnp.int32, L)
    cnt_one = jnp.where(lane == 0, 1.0, 0.0).astype(jnp.float32)

    def cinit(i, _):
        hrows0[i, pl.ds(D, L)] = cnt_one
        hrows1[i, pl.ds(D, L)] = cnt_one
        return 0

    lax.fori_loop(0, B, cinit, 0)

    def issue_idx(k, b):
        base = pl.multiple_of(ebase + k * B, 8)
        pltpu.async_copy(src_hbm.at[pl.ds(base, B)], idx_s[b], sem_i[b])
        pltpu.async_copy(tgt_hbm.at[pl.ds(base, B)], idx_t[b], sem_i[b])

    def wait_idx(b):
        pltpu.make_async_copy(src_hbm.at[pl.ds(0, B)], idx_s[b],
                              sem_i[b]).wait()
        pltpu.make_async_copy(tgt_hbm.at[pl.ds(0, B)], idx_t[b],
                              sem_i[b]).wait()

    ABLATE_GATHER = False

    def issue_gather(b):
        if ABLATE_GATHER:
            return
        pltpu.async_copy(vp_hbm.at[idx_s[b]], rows_s[b], sem_g[b])
        pltpu.async_copy(v_hbm.at[idx_t[b]], rows_t[b], sem_g[b])

    def wait_gather(b):
        if ABLATE_GATHER:
            return
        pltpu.make_async_copy(vp_hbm.at[idx_s[b]], rows_s[b], sem_g[b]).wait()
        pltpu.make_async_copy(v_hbm.at[idx_t[b]], rows_t[b], sem_g[b]).wait()

    def copy_idx(b):
        for o in (0, 16, B - 16):
            idx_c[b][pl.ds(o, L)] = idx_s[b][pl.ds(o, L)]

    ABLATE_SCATTER = False

    def issue_scatter(b):
        if not ABLATE_SCATTER:
            pltpu.async_copy(hrows[b], h_sp.at[idx_c[b]], sem_c[b], add=True)

    def wait_scatter(b):
        if not ABLATE_SCATTER:
            pltpu.make_async_copy(hrows[b], h_sp.at[idx_c[b]],
                                  sem_c[b]).wait()

    ABLATE_COMPUTE = False

    def compute(b):
        if ABLATE_COMPUTE:
            return

        @plsc.parallel_loop(0, B, unroll=8)
        def edge(i):
            for r in range(D // L):
                sl = pl.ds(r * L, L)
                hrows[b][i, sl] = jnp.maximum(
                    rows_s[b][i, sl] - rows_t[b][i, sl], 0.0)

    issue_idx(0, 0)
    issue_idx(1, 1)
    wait_idx(0)
    issue_gather(0)
    plsc.subcore_barrier()

    def outer(ko, _):
        for bb in range(2):
            k = ko * 2 + bb

            @pl.when(ko > 0)
            def _():
                wait_scatter(bb)

            copy_idx(bb)
            wait_gather(bb)

            @pl.when(ko < NCHUNK // 2 - 1)
            def _():
                issue_idx(k + 2, bb)

            if bb == 0:
                wait_idx(1)
                issue_gather(1)
            else:
                @pl.when(ko < NCHUNK // 2 - 1)
                def _():
                    wait_idx(0)
                    issue_gather(0)

            compute(bb)
            issue_scatter(bb)
        return 0

    lax.fori_loop(0, NCHUNK // 2, outer, 0)
    wait_scatter(0)
    wait_scatter(1)
    plsc.subcore_barrier()

    @pl.when(s == 0)
    def _():
        pltpu.sync_copy(h_sp,
                        hc_hbm.at[pl.ds(c * N, N), pl.ds(0, DC)])


_edge = functools.partial(
    pl.kernel,
    out_type=jax.ShapeDtypeStruct((NC * N, DCP), jnp.float32),
    mesh=_mesh,
    scratch_types=[
        pltpu.VMEM_SHARED((N, DC), jnp.float32),
        pltpu.VMEM((B,), jnp.int32),
        pltpu.VMEM((B,), jnp.int32),
        pltpu.VMEM((B,), jnp.int32),
        pltpu.VMEM((B,), jnp.int32),
        pltpu.VMEM((B,), jnp.int32),
        pltpu.VMEM((B,), jnp.int32),
        pltpu.VMEM((B, D), jnp.float32),
        pltpu.VMEM((B, D), jnp.float32),
        pltpu.VMEM((B, D), jnp.float32),
        pltpu.VMEM((B, D), jnp.float32),
        pltpu.VMEM((B, DC), jnp.float32),
        pltpu.VMEM((B, DC), jnp.float32),
        pltpu.VMEM((ZR, DC), jnp.float32),
        pltpu.SemaphoreType.DMA,
        pltpu.SemaphoreType.DMA,
        pltpu.SemaphoreType.DMA,
        pltpu.SemaphoreType.DMA,
        pltpu.SemaphoreType.DMA,
        pltpu.SemaphoreType.DMA,
    ],
    compiler_params=pltpu.CompilerParams(
        needs_layout_passes=False, use_tc_tiling_on_sc=False),
)(_edge_body)


BN1 = 1024
BN2 = 1000


def _mm1_body(x_ref, g_ref, w1x_ref, w1g_ref, b1_ref, vp_ref, v_ref):
    v = (
        jnp.dot(x_ref[...], w1x_ref[...], preferred_element_type=jnp.float32)
        + lax.dot_general(
            g_ref[0], w1g_ref[...],
            dimension_numbers=(((0,), (0,)), ((), ())),
            preferred_element_type=jnp.float32))
    v_ref[...] = v
    vp_ref[...] = v + b1_ref[...]


def _mm1(x_pad, g3, w1x, w1g_p, b1_row):
    return pl.pallas_call(
        _mm1_body,
        grid=(NV // BN1,),
        in_specs=[
            pl.BlockSpec((BN1, D), lambda j: (j, 0)),
            pl.BlockSpec((1, 16, BN1), lambda j: (j, 0, 0)),
            pl.BlockSpec((D, D), lambda j: (0, 0)),
            pl.BlockSpec((16, D), lambda j: (0, 0)),
            pl.BlockSpec((1, D), lambda j: (0, 0)),
        ],
        out_specs=[
            pl.BlockSpec((BN1, D), lambda j: (j, 0)),
            pl.BlockSpec((BN1, D), lambda j: (j, 0)),
        ],
        out_shape=[
            jax.ShapeDtypeStruct((NV, D), jnp.float32),
            jax.ShapeDtypeStruct((NV, D), jnp.float32),
        ],
    )(x_pad, g3, w1x, w1g_p, b1_row)


def _mm2_body(h0_ref, h1_ref, w2_ref, b2_ref, o_ref):
    a = h0_ref[:, :D] + h1_ref[:, :D]
    cnt = h0_ref[:, D:D + 1] + h1_ref[:, D:D + 1]
    o_ref[...] = (
        jnp.dot(a, w2_ref[...], preferred_element_type=jnp.float32)
        + cnt * b2_ref[...])


def _mm2(hc, w2, b2_row):
    return pl.pallas_call(
        _mm2_body,
        grid=(N // BN2,),
        in_specs=[
            pl.BlockSpec((BN2, DCP), lambda j: (j, 0)),
            pl.BlockSpec((BN2, DCP), lambda j: (j + N // BN2, 0)),
            pl.BlockSpec((D, D), lambda j: (0, 0)),
            pl.BlockSpec((1, D), lambda j: (0, 0)),
        ],
        out_specs=pl.BlockSpec((BN2, D), lambda j: (j, 0)),
        out_shape=jax.ShapeDtypeStruct((N, D), jnp.float32),
    )(hc, hc, w2, b2_row)


def kernel(x, points, triangles, triangle_edges, W1, b1, W2, b2):
    tri = triangles.astype(jnp.int32)
    tpad = jnp.zeros((N_PAD - N,), jnp.int32)
    t0 = jnp.concatenate([tri[:, 0], tpad])
    t1 = jnp.concatenate([tri[:, 1], tpad])
    t2 = jnp.concatenate([tri[:, 2], tpad])
    px, py, pz = points[:, 0], points[:, 1], points[:, 2]
    src = triangle_edges[0].astype(jnp.int32)
    tgt = triangle_edges[1].astype(jnp.int32)
    w1g_p = jnp.concatenate(
        [W1[:9], jnp.zeros((16 - 9, D), W1.dtype)], axis=0)
    w1x = W1[9:]
    x_pad = jnp.concatenate([x, jnp.zeros((NV - N, D), x.dtype)], axis=0)

    g_t = _geom(px, py, pz, t0, t1, t2)
    g3 = jnp.moveaxis(g_t.reshape(16, N_PAD // BN1, BN1), 1, 0)
    vp, v = _mm1(x_pad, g3, w1x, w1g_p, b1.reshape(1, D))
    hc = _edge(vp, v, src, tgt)
    return _mm2(hc, W2, b2.reshape(1, D))


# speedup vs baseline: 1.2912x; 1.2330x over previous
import functools

import jax
import jax.numpy as jnp
from jax import lax
from jax.experimental import pallas as pl
from jax.experimental.pallas import tpu as pltpu, tpu_sc as plsc

N = 10000
P = 5000
E = 320000
D = 128

NC, NS, L = 2, 16, 16
NW = NC * NS
TRI_W = 384
N_PAD = NW * TRI_W
NV = 10240
EPW = E // NW
B = 40
NCHUNK = EPW // B
DC = D + 16
DCP = 256
RPT = N // NS
ZR = 25

_mesh = plsc.VectorSubcoreMesh(
    core_axis_name="c", subcore_axis_name="s", num_cores=NC, num_subcores=NS)


def _geom_body(px_hbm, py_hbm, pz_hbm, t0_hbm, t1_hbm, t2_hbm, g_hbm,
               px_v, py_v, pz_v, t0_v, t1_v, t2_v, gt_v):
    c = lax.axis_index("c")
    s = lax.axis_index("s")
    wid = s * NC + c
    base = pl.multiple_of(wid * TRI_W, 128)
    pltpu.sync_copy(px_hbm, px_v)
    pltpu.sync_copy(py_hbm, py_v)
    pltpu.sync_copy(pz_hbm, pz_v)
    pltpu.sync_copy(t0_hbm.at[pl.ds(base, TRI_W)], t0_v)
    pltpu.sync_copy(t1_hbm.at[pl.ds(base, TRI_W)], t1_v)
    pltpu.sync_copy(t2_hbm.at[pl.ds(base, TRI_W)], t2_v)

    zero = jnp.zeros((L,), jnp.float32)
    third = jnp.full((L,), 1.0 / 3.0, jnp.float32)
    coords = (px_v, py_v, pz_v)
    tcols = (t0_v, t1_v, t2_v)

    def group(i, _):
        sl = pl.ds(i * L, L)
        idx = [tcols[j][sl] for j in range(3)]
        for cc in range(3):
            p0 = plsc.load_gather(coords[cc], [idx[0]])
            p1 = plsc.load_gather(coords[cc], [idx[1]])
            p2 = plsc.load_gather(coords[cc], [idx[2]])
            e0 = p0 - p1
            e1 = p0 - p2
            e2 = p1 - p2
            gt_v[0 + cc, sl] = jnp.minimum(jnp.minimum(e0, e1), e2)
            gt_v[3 + cc, sl] = jnp.maximum(jnp.maximum(e0, e1), e2)
            gt_v[6 + cc, sl] = (p0 + p1 + p2) * third
        for r in range(9, 16):
            gt_v[r, sl] = zero
        return 0

    lax.fori_loop(0, TRI_W // L, group, 0)
    pltpu.sync_copy(gt_v, g_hbm.at[:, pl.ds(base, TRI_W)])


_geom = functools.partial(
    pl.kernel,
    out_type=jax.ShapeDtypeStruct((16, N_PAD), jnp.float32),
    mesh=_mesh,
    scratch_types=[
        pltpu.VMEM((P,), jnp.float32),
        pltpu.VMEM((P,), jnp.float32),
        pltpu.VMEM((P,), jnp.float32),
        pltpu.VMEM((TRI_W,), jnp.int32),
        pltpu.VMEM((TRI_W,), jnp.int32),
        pltpu.VMEM((TRI_W,), jnp.int32),
        pltpu.VMEM((16, TRI_W), jnp.float32),
    ],
    compiler_params=pltpu.CompilerParams(needs_layout_passes=False),
)(_geom_body)


def _edge_body(vp_hbm, v_hbm, src_hbm, tgt_hbm, hc_hbm,
               h_sp,
               idx_s0, idx_s1, idx_t0, idx_t1, idx_c0, idx_c1,
               rows_s0, rows_s1, rows_t0, rows_t1, hrows0, hrows1,
               zbuf,
               sem_i0, sem_i1, sem_g0, sem_g1, sem_c0, sem_c1):
    c = lax.axis_index("c")
    s = lax.axis_index("s")
    wid = s * NC + c
    ebase = wid * EPW

    idx_s = (idx_s0, idx_s1)
    idx_t = (idx_t0, idx_t1)
    idx_c = (idx_c0, idx_c1)
    rows_s = (rows_s0, rows_s1)
    rows_t = (rows_t0, rows_t1)
    hrows = (hrows0, hrows1)
    sem_i = (sem_i0, sem_i1)
    sem_g = (sem_g0, sem_g1)
    sem_c = (sem_c0, sem_c1)

    zero = jnp.zeros((L,), jnp.float32)

    def zrow(i, _):
        for r in range(DC // L):
            zbuf[i, pl.ds(r * L, L)] = zero
        return 0

    lax.fori_loop(0, ZR, zrow, 0)
    for m in range(RPT // ZR):
        pltpu.sync_copy(zbuf, h_sp.at[pl.ds(s * RPT + m * ZR, ZR)])

    lane = lax.iota(jnp.int32, L)
    cnt_one = jnp.where(lane == 0, 1.0, 0.0).astype(jnp.float32)

    def cinit(i, _):
        hrows0[i, pl.ds(D, L)] = cnt_one
        hrows1[i, pl.ds(D, L)] = cnt_one
        return 0

    lax.fori_loop(0, B, cinit, 0)

    def issue_idx(k, b):
        base = pl.multiple_of(ebase + k * B, 8)
        pltpu.async_copy(src_hbm.at[pl.ds(base, B)], idx_s[b], sem_i[b])
        pltpu.async_copy(tgt_hbm.at[pl.ds(base, B)], idx_t[b], sem_i[b])

    def wait_idx(b):
        pltpu.make_async_copy(src_hbm.at[pl.ds(0, B)], idx_s[b],
                              sem_i[b]).wait()
        pltpu.make_async_copy(tgt_hbm.at[pl.ds(0, B)], idx_t[b],
                              sem_i[b]).wait()

    ABLATE_GATHER = False

    def issue_gather(b):
        if ABLATE_GATHER:
            return
        pltpu.async_copy(vp_hbm.at[idx_s[b]], rows_s[b], sem_g[b])
        pltpu.async_copy(v_hbm.at[idx_t[b]], rows_t[b], sem_g[b])

    def wait_gather(b):
        if ABLATE_GATHER:
            return
        pltpu.make_async_copy(vp_hbm.at[idx_s[b]], rows_s[b], sem_g[b]).wait()
        pltpu.make_async_copy(v_hbm.at[idx_t[b]], rows_t[b], sem_g[b]).wait()

    def copy_idx(b):
        for o in (0, 16, B - 16):
            idx_c[b][pl.ds(o, L)] = idx_s[b][pl.ds(o, L)]

    ABLATE_SCATTER = False

    def issue_scatter(b):
        if not ABLATE_SCATTER:
            pltpu.async_copy(hrows[b], h_sp.at[idx_c[b]], sem_c[b], add=True)

    def wait_scatter(b):
        if not ABLATE_SCATTER:
            pltpu.make_async_copy(hrows[b], h_sp.at[idx_c[b]],
                                  sem_c[b]).wait()

    ABLATE_COMPUTE = False

    def compute(b):
        if ABLATE_COMPUTE:
            return

        @plsc.parallel_loop(0, B, unroll=8)
        def edge(i):
            for r in range(D // L):
                sl = pl.ds(r * L, L)
                hrows[b][i, sl] = jnp.maximum(
                    rows_s[b][i, sl] - rows_t[b][i, sl], 0.0)

    issue_idx(0, 0)
    issue_idx(1, 1)
    wait_idx(0)
    issue_gather(0)
    plsc.subcore_barrier()

    def outer(ko, _):
        for bb in range(2):
            k = ko * 2 + bb

            @pl.when(ko > 0)
            def _():
                wait_scatter(bb)

            copy_idx(bb)

            if bb == 0:
                wait_idx(1)
                issue_gather(1)
            else:
                @pl.when(ko < NCHUNK // 2 - 1)
                def _():
                    wait_idx(0)
                    issue_gather(0)

            wait_gather(bb)

            @pl.when(ko < NCHUNK // 2 - 1)
            def _():
                issue_idx(k + 2, bb)

            compute(bb)
            issue_scatter(bb)
        return 0

    lax.fori_loop(0, NCHUNK // 2, outer, 0)
    wait_scatter(0)
    wait_scatter(1)
    plsc.subcore_barrier()

    @pl.when(s == 0)
    def _():
        pltpu.sync_copy(h_sp,
                        hc_hbm.at[pl.ds(c * N, N), pl.ds(0, DC)])


_edge = functools.partial(
    pl.kernel,
    out_type=jax.ShapeDtypeStruct((NC * N, DCP), jnp.float32),
    mesh=_mesh,
    scratch_types=[
        pltpu.VMEM_SHARED((N, DC), jnp.float32),
        pltpu.VMEM((B,), jnp.int32),
        pltpu.VMEM((B,), jnp.int32),
        pltpu.VMEM((B,), jnp.int32),
        pltpu.VMEM((B,), jnp.int32),
        pltpu.VMEM((B,), jnp.int32),
        pltpu.VMEM((B,), jnp.int32),
        pltpu.VMEM((B, D), jnp.float32),
        pltpu.VMEM((B, D), jnp.float32),
        pltpu.VMEM((B, D), jnp.float32),
        pltpu.VMEM((B, D), jnp.float32),
        pltpu.VMEM((B, DC), jnp.float32),
        pltpu.VMEM((B, DC), jnp.float32),
        pltpu.VMEM((ZR, DC), jnp.float32),
        pltpu.SemaphoreType.DMA,
        pltpu.SemaphoreType.DMA,
        pltpu.SemaphoreType.DMA,
        pltpu.SemaphoreType.DMA,
        pltpu.SemaphoreType.DMA,
        pltpu.SemaphoreType.DMA,
    ],
    compiler_params=pltpu.CompilerParams(
        needs_layout_passes=False, use_tc_tiling_on_sc=False),
)(_edge_body)


BN1 = 1024
BN2 = 1000


def _mm1_body(x_ref, g_ref, w1x_ref, w1g_ref, b1_ref, vp_ref, v_ref):
    v = (
        jnp.dot(x_ref[...], w1x_ref[...], preferred_element_type=jnp.float32)
        + lax.dot_general(
            g_ref[0], w1g_ref[...],
            dimension_numbers=(((0,), (0,)), ((), ())),
            preferred_element_type=jnp.float32))
    v_ref[...] = v
    vp_ref[...] = v + b1_ref[...]


def _mm1(x_in, g3, w1x, w1g_p, b1_row):
    return pl.pallas_call(
        _mm1_body,
        grid=(N // BN2,),
        in_specs=[
            pl.BlockSpec((BN2, D), lambda j: (j, 0)),
            pl.BlockSpec((1, 16, BN2), lambda j: (j, 0, 0)),
            pl.BlockSpec((D, D), lambda j: (0, 0)),
            pl.BlockSpec((16, D), lambda j: (0, 0)),
            pl.BlockSpec((1, D), lambda j: (0, 0)),
        ],
        out_specs=[
            pl.BlockSpec((BN2, D), lambda j: (j, 0)),
            pl.BlockSpec((BN2, D), lambda j: (j, 0)),
        ],
        out_shape=[
            jax.ShapeDtypeStruct((N, D), jnp.float32),
            jax.ShapeDtypeStruct((N, D), jnp.float32),
        ],
    )(x_in, g3, w1x, w1g_p, b1_row)


def _mm2_body(h0_ref, h1_ref, w2_ref, b2_ref, o_ref):
    a = h0_ref[:, :D] + h1_ref[:, :D]
    cnt = h0_ref[:, D:D + 1] + h1_ref[:, D:D + 1]
    o_ref[...] = (
        jnp.dot(a, w2_ref[...], preferred_element_type=jnp.float32)
        + cnt * b2_ref[...])


def _mm2(hc, w2, b2_row):
    return pl.pallas_call(
        _mm2_body,
        grid=(N // BN2,),
        in_specs=[
            pl.BlockSpec((BN2, DCP), lambda j: (j, 0)),
            pl.BlockSpec((BN2, DCP), lambda j: (j + N // BN2, 0)),
            pl.BlockSpec((D, D), lambda j: (0, 0)),
            pl.BlockSpec((1, D), lambda j: (0, 0)),
        ],
        out_specs=pl.BlockSpec((BN2, D), lambda j: (j, 0)),
        out_shape=jax.ShapeDtypeStruct((N, D), jnp.float32),
    )(hc, hc, w2, b2_row)


def kernel(x, points, triangles, triangle_edges, W1, b1, W2, b2):
    tri = triangles.astype(jnp.int32)
    tpad = jnp.zeros((N_PAD - N,), jnp.int32)
    t0 = jnp.concatenate([tri[:, 0], tpad])
    t1 = jnp.concatenate([tri[:, 1], tpad])
    t2 = jnp.concatenate([tri[:, 2], tpad])
    px, py, pz = points[:, 0], points[:, 1], points[:, 2]
    src = triangle_edges[0].astype(jnp.int32)
    tgt = triangle_edges[1].astype(jnp.int32)
    w1g_p = jnp.concatenate(
        [W1[:9], jnp.zeros((16 - 9, D), W1.dtype)], axis=0)
    w1x = W1[9:]
    g_t = _geom(px, py, pz, t0, t1, t2)
    g3 = jnp.moveaxis(g_t[:, :N].reshape(16, N // BN2, BN2), 1, 0)
    vp, v = _mm1(x, g3, w1x, w1g_p, b1.reshape(1, D))
    hc = _edge(vp, v, src, tgt)
    return _mm2(hc, W2, b2.reshape(1, D))
